# baseline (speedup 1.0000x reference)
import math

import jax
import jax.numpy as jnp
from jax.experimental import pallas as pl
from jax.experimental.pallas import tpu as pltpu

N = 2048
D = 256
H = 4
HD = 64
KN = 5
RB = 256
NB = N // RB
NEG = -1e30
HI = jax.lax.Precision.HIGHEST


def _rowmax(x):
    n = x.shape[1]
    m = x[:, 0:128]
    for c in range(1, n // 128):
        m = jnp.maximum(m, x[:, c * 128:(c + 1) * 128])
    return jnp.max(m, axis=1, keepdims=True)


def _ln(x, g, b, eps=1e-3):
    m = jnp.mean(x, axis=-1, keepdims=True)
    v = jnp.mean((x - m) * (x - m), axis=-1, keepdims=True)
    return (x - m) / jnp.sqrt(v + eps) * g + b


def _ident(g):
    return jnp.where(jax.lax.broadcasted_iota(jnp.int32, (g, g), 0) ==
                     jax.lax.broadcasted_iota(jnp.int32, (g, g), 1), 1.0, 0.0)


def _qkv_body(lv_ref, wq_ref, wk_ref, wv_ref, bq_ref, bk_ref, bv_ref,
              q_ref, k_ref, v_ref):
    lv = lv_ref[...]
    q_ref[...] = jnp.dot(lv, wq_ref[...], preferred_element_type=jnp.float32) + bq_ref[...]
    k_ref[...] = jnp.dot(lv, wk_ref[...], preferred_element_type=jnp.float32) + bk_ref[...]
    v_ref[...] = jnp.dot(lv, wv_ref[...], preferred_element_type=jnp.float32) + bv_ref[...]


def _attn_topk_body(q_ref, k_ref, topk_ref):
    q = q_ref[...]
    k = k_ref[...]
    scale = 1.0 / math.sqrt(HD)
    acc = jnp.zeros((RB, N), jnp.float32)
    for h in range(H):
        s = jax.lax.dot_general(q[:, h * HD:(h + 1) * HD], k[:, h * HD:(h + 1) * HD],
                                (((1,), (1,)), ((), ())),
                                preferred_element_type=jnp.float32) * scale
        s = s - _rowmax(s)
        e = jnp.exp(s)
        acc = acc + e / jnp.sum(e, axis=1, keepdims=True)
    am = acc * (1.0 / H)
    col = jax.lax.broadcasted_iota(jnp.int32, (RB, N), 1).astype(jnp.float32)
    picks = []
    for _ in range(KN):
        vmax = _rowmax(am)
        imax = _rowmax(jnp.where(am >= vmax, col, -1.0))
        picks.append(imax)
        am = jnp.where(col == imax, NEG, am)
    topk_ref[...] = jnp.concatenate(picks[::-1], axis=1)


def _tail_body(topk_ref, lv_ref, q_ref, k_ref, v_ref,
               wo_ref, bo_ref, ln1g_ref, ln1b_ref,
               wg0_ref, bg0_ref, lng0_ref, lnb0_ref,
               wg1_ref, bg1_ref, lng1_ref, lnb1_ref,
               w1m_ref, b1m_ref, w2m_ref, b2m_ref,
               w1y_ref, b1y_ref, w2y_ref, b2y_ref,
               w1r_ref, b1r_ref, w2r_ref, b2r_ref,
               o1_ref, o2_ref, o3_ref):
    topk = topk_ref[...]
    scale = 1.0 / math.sqrt(HD)
    iota_n = jax.lax.broadcasted_iota(jnp.int32, (1, N), 1).astype(jnp.float32)
    iota5 = jax.lax.broadcasted_iota(jnp.int32, (1, KN), 1)

    def gatherc(e_col_i32):
        rowid = (e_col_i32 // KN).astype(jnp.float32)
        colid = e_col_i32 % KN
        onehot = jnp.where(rowid == iota_n, 1.0, 0.0)
        vals5 = jnp.dot(onehot, topk, preferred_element_type=jnp.float32,
                        precision=HI)
        return jnp.sum(jnp.where(colid == iota5, vals5, 0.0),
                       axis=1, keepdims=True)

    r15 = jax.lax.broadcasted_iota(jnp.int32, (15, 1), 0)
    e2 = (r15 // KN) + (r15 % KN) * N
    s2 = gatherc(e2)
    c3 = jax.lax.broadcasted_iota(jnp.int32, (3, 1), 0).astype(jnp.float32)
    S1 = jnp.concatenate([c3, s2], axis=0)
    r90 = jax.lax.broadcasted_iota(jnp.int32, (90, 1), 0)
    rep18 = jnp.where((r90 // KN) == jax.lax.broadcasted_iota(jnp.int32, (1, 18), 1),
                      1.0, 0.0)
    S1rep = jnp.dot(rep18, S1, preferred_element_type=jnp.float32, precision=HI)
    e1 = S1rep.astype(jnp.int32) + (r90 % KN) * N
    s1 = gatherc(e1)

    sall = jnp.concatenate([s2, s1], axis=0)
    sall_row = jax.lax.dot_general(sall, _ident(105), (((0,), (0,)), ((), ())),
                                   preferred_element_type=jnp.float32,
                                   precision=HI)
    cnt = jnp.zeros((N, 105), jnp.float32)
    for j in range(KN):
        cnt = cnt + jnp.where(topk[:, j:j + 1] == sall_row, 1.0, 0.0)
    ones_row = jnp.zeros((1, N), jnp.float32) + 1.0
    deg_row = jnp.dot(ones_row, cnt, preferred_element_type=jnp.float32,
                      precision=HI)
    norm_row = jax.lax.rsqrt(5.0 * deg_row)
    norm_col = jax.lax.dot_general(_ident(105), norm_row, (((1,), (1,)), ((), ())),
                                   preferred_element_type=jnp.float32,
                                   precision=HI)
    norm2 = norm_col[0:15, :]
    norm1 = norm_col[15:105, :]

    S0 = jnp.concatenate([S1, s1], axis=0)
    onehot0 = jnp.where(S0 == iota_n, 1.0, 0.0)
    lv_sel = jnp.dot(onehot0, lv_ref[...], preferred_element_type=jnp.float32,
                     precision=HI)
    q_sel = jnp.dot(onehot0, q_ref[...], preferred_element_type=jnp.float32,
                    precision=HI)

    k_all = k_ref[...]
    v_all = v_ref[...]
    ctxs = []
    for h in range(H):
        s = jax.lax.dot_general(q_sel[:, h * HD:(h + 1) * HD], k_all[:, h * HD:(h + 1) * HD],
                                (((1,), (1,)), ((), ())),
                                preferred_element_type=jnp.float32) * scale
        s = s - _rowmax(s)
        e = jnp.exp(s)
        p = e / jnp.sum(e, axis=1, keepdims=True)
        ctxs.append(jnp.dot(p, v_all[:, h * HD:(h + 1) * HD],
                            preferred_element_type=jnp.float32))
    ctx = jnp.concatenate(ctxs, axis=1)
    mha = jnp.dot(ctx, wo_ref[...], preferred_element_type=jnp.float32) + bo_ref[...]
    x0 = _ln(lv_sel + mha, ln1g_ref[...], ln1b_ref[...])

    red18 = jnp.where(jax.lax.broadcasted_iota(jnp.int32, (18, 90), 0) ==
                      (jax.lax.broadcasted_iota(jnp.int32, (18, 90), 1) // KN),
                      1.0, 0.0)
    agg1 = jnp.dot(red18, x0[18:108, :] * norm1, preferred_element_type=jnp.float32,
                   precision=HI)
    x1 = _ln(x0[0:18, :] + jnp.dot(agg1, wg0_ref[...], preferred_element_type=jnp.float32) + bg0_ref[...],
             lng0_ref[...], lnb0_ref[...])

    red3 = jnp.where(jax.lax.broadcasted_iota(jnp.int32, (3, 15), 0) ==
                     (jax.lax.broadcasted_iota(jnp.int32, (3, 15), 1) // KN),
                     1.0, 0.0)
    agg2 = jnp.dot(red3, x1[3:18, :] * norm2, preferred_element_type=jnp.float32,
                   precision=HI)
    x2 = _ln(x1[0:3, :] + jnp.dot(agg2, wg1_ref[...], preferred_element_type=jnp.float32) + bg1_ref[...],
             lng1_ref[...], lnb1_ref[...])

    for row, (w1, b1, w2, b2, out) in enumerate((
            (w1m_ref, b1m_ref, w2m_ref, b2m_ref, o1_ref),
            (w1y_ref, b1y_ref, w2y_ref, b2y_ref, o2_ref),
            (w1r_ref, b1r_ref, w2r_ref, b2r_ref, o3_ref))):
        hdn = jnp.maximum(
            jnp.dot(x2[row:row + 1, :], w1[...], preferred_element_type=jnp.float32) + b1[...],
            0.0)
        out[...] = jnp.dot(hdn, w2[...], preferred_element_type=jnp.float32) + b2[...]


@jax.jit
def kernel(latent_vectors, Wq, bq, Wk, bk, Wv, bv, Wo, bo, ln1_g, ln1_b,
           Wg0, bg0, lng0, lnb0, Wg1, bg1, lng1, lnb1,
           W1_mrna, b1_mrna, W2_mrna, b2_mrna,
           W1_methylation, b1_methylation, W2_methylation, b2_methylation,
           W1_mirna, b1_mirna, W2_mirna, b2_mirna):
    q_all, k_all, v_all = pl.pallas_call(
        _qkv_body,
        out_shape=[jax.ShapeDtypeStruct((N, D), jnp.float32)] * 3,
    )(latent_vectors,
      Wq.reshape(D, D), Wk.reshape(D, D), Wv.reshape(D, D),
      bq.reshape(1, D), bk.reshape(1, D), bv.reshape(1, D))

    topk = pl.pallas_call(
        _attn_topk_body,
        grid=(NB,),
        in_specs=[pl.BlockSpec((RB, D), lambda i: (i, 0)),
                  pl.BlockSpec((N, D), lambda i: (0, 0))],
        out_specs=pl.BlockSpec((RB, KN), lambda i: (i, 0)),
        out_shape=jax.ShapeDtypeStruct((N, KN), jnp.float32),
        compiler_params=pltpu.CompilerParams(
            dimension_semantics=("parallel",)),
    )(q_all, k_all)

    o1, o2, o3 = pl.pallas_call(
        _tail_body,
        out_shape=[jax.ShapeDtypeStruct((1, 1000), jnp.float32),
                   jax.ShapeDtypeStruct((1, 2000), jnp.float32),
                   jax.ShapeDtypeStruct((1, 500), jnp.float32)],
    )(topk, latent_vectors, q_all, k_all, v_all,
      Wo.reshape(D, D), bo.reshape(1, D), ln1_g.reshape(1, D), ln1_b.reshape(1, D),
      Wg0, bg0.reshape(1, D), lng0.reshape(1, D), lnb0.reshape(1, D),
      Wg1, bg1.reshape(1, D), lng1.reshape(1, D), lnb1.reshape(1, D),
      W1_mrna, b1_mrna.reshape(1, -1), W2_mrna, b2_mrna.reshape(1, -1),
      W1_methylation, b1_methylation.reshape(1, -1), W2_methylation, b2_methylation.reshape(1, -1),
      W1_mirna, b1_mirna.reshape(1, -1), W2_mirna, b2_mirna.reshape(1, -1))
    return (o1.reshape(-1), o2.reshape(-1), o3.reshape(-1))


# speedup vs baseline: 1.1179x; 1.1179x over previous
import math

import jax
import jax.numpy as jnp
from jax.experimental import pallas as pl
from jax.experimental.pallas import tpu as pltpu

N = 2048
D = 256
H = 4
HD = 64
KN = 5
RB = 512
NB = N // RB
NEG = -1e30
HI = jax.lax.Precision.HIGHEST


def _rowmax(x):
    n = x.shape[1]
    m = x[:, 0:128]
    for c in range(1, n // 128):
        m = jnp.maximum(m, x[:, c * 128:(c + 1) * 128])
    return jnp.max(m, axis=1, keepdims=True)


def _ln(x, g, b, eps=1e-3):
    m = jnp.mean(x, axis=-1, keepdims=True)
    v = jnp.mean((x - m) * (x - m), axis=-1, keepdims=True)
    return (x - m) / jnp.sqrt(v + eps) * g + b


def _ident(g):
    return jnp.where(jax.lax.broadcasted_iota(jnp.int32, (g, g), 0) ==
                     jax.lax.broadcasted_iota(jnp.int32, (g, g), 1), 1.0, 0.0)


def _body(lv_ref, wq_ref, wk_ref, wv_ref, bq_ref, bk_ref, bv_ref,
          wo_ref, bo_ref, ln1g_ref, ln1b_ref,
          wg0_ref, bg0_ref, lng0_ref, lnb0_ref,
          wg1_ref, bg1_ref, lng1_ref, lnb1_ref,
          w1m_ref, b1m_ref, w2m_ref, b2m_ref,
          w1y_ref, b1y_ref, w2y_ref, b2y_ref,
          w1r_ref, b1r_ref, w2r_ref, b2r_ref,
          o1_ref, o2_ref, o3_ref,
          qkv_s, topk_s):
    i = pl.program_id(0)
    scale = 1.0 / math.sqrt(HD)

    @pl.when(i == 0)
    def _qkv():
        lv = lv_ref[...]
        qkv_s[:, 0:D] = jnp.dot(lv, wq_ref[...], preferred_element_type=jnp.float32) + bq_ref[...]
        qkv_s[:, D:2 * D] = jnp.dot(lv, wk_ref[...], preferred_element_type=jnp.float32) + bk_ref[...]
        qkv_s[:, 2 * D:3 * D] = jnp.dot(lv, wv_ref[...], preferred_element_type=jnp.float32) + bv_ref[...]

    @pl.when((i >= 1) & (i <= NB))
    def _attn():
        r0 = jnp.maximum(i - 1, 0) * RB
        q = qkv_s[pl.ds(r0, RB), 0:D]
        k = qkv_s[:, D:2 * D]
        acc = jnp.zeros((RB, N), jnp.float32)
        for h in range(H):
            s = jax.lax.dot_general(q[:, h * HD:(h + 1) * HD], k[:, h * HD:(h + 1) * HD],
                                    (((1,), (1,)), ((), ())),
                                    preferred_element_type=jnp.float32) * scale
            s = s - _rowmax(s)
            e = jnp.exp(s)
            acc = acc + e / jnp.sum(e, axis=1, keepdims=True)
        am = acc * (1.0 / H)
        col = jax.lax.broadcasted_iota(jnp.int32, (RB, N), 1).astype(jnp.float32)
        picks = []
        for _ in range(KN):
            vmax = _rowmax(am)
            imax = _rowmax(jnp.where(am >= vmax, col, -1.0))
            picks.append(imax)
            am = jnp.where(col == imax, NEG, am)
        topk_s[pl.ds(r0, RB), :] = jnp.concatenate(picks[::-1], axis=1)

    @pl.when(i == NB + 1)
    def _tail():
        topk = topk_s[...]
        iota_n = jax.lax.broadcasted_iota(jnp.int32, (1, N), 1).astype(jnp.float32)
        iota5 = jax.lax.broadcasted_iota(jnp.int32, (1, KN), 1)

        def gatherc(e_col_i32):
            rowid = (e_col_i32 // KN).astype(jnp.float32)
            colid = e_col_i32 % KN
            onehot = jnp.where(rowid == iota_n, 1.0, 0.0)
            vals5 = jnp.dot(onehot, topk, preferred_element_type=jnp.float32,
                            precision=HI)
            return jnp.sum(jnp.where(colid == iota5, vals5, 0.0),
                           axis=1, keepdims=True)

        r15 = jax.lax.broadcasted_iota(jnp.int32, (15, 1), 0)
        e2 = (r15 // KN) + (r15 % KN) * N
        s2 = gatherc(e2)
        c3 = jax.lax.broadcasted_iota(jnp.int32, (3, 1), 0).astype(jnp.float32)
        S1 = jnp.concatenate([c3, s2], axis=0)
        r90 = jax.lax.broadcasted_iota(jnp.int32, (90, 1), 0)
        rep18 = jnp.where((r90 // KN) == jax.lax.broadcasted_iota(jnp.int32, (1, 18), 1),
                          1.0, 0.0)
        S1rep = jnp.dot(rep18, S1, preferred_element_type=jnp.float32, precision=HI)
        e1 = S1rep.astype(jnp.int32) + (r90 % KN) * N
        s1 = gatherc(e1)

        sall = jnp.concatenate([s2, s1], axis=0)
        sall_row = jax.lax.dot_general(sall, _ident(105), (((0,), (0,)), ((), ())),
                                       preferred_element_type=jnp.float32,
                                       precision=HI)
        cnt = jnp.zeros((N, 105), jnp.float32)
        for j in range(KN):
            cnt = cnt + jnp.where(topk[:, j:j + 1] == sall_row, 1.0, 0.0)
        ones_row = jnp.zeros((1, N), jnp.float32) + 1.0
        deg_row = jnp.dot(ones_row, cnt, preferred_element_type=jnp.float32)
        norm_row = jax.lax.rsqrt(5.0 * deg_row)
        norm_col = jax.lax.dot_general(_ident(105), norm_row, (((1,), (1,)), ((), ())),
                                       preferred_element_type=jnp.float32,
                                       precision=HI)
        norm2 = norm_col[0:15, :]
        norm1 = norm_col[15:105, :]

        S0 = jnp.concatenate([S1, s1], axis=0)
        onehot0 = jnp.where(S0 == iota_n, 1.0, 0.0)
        lv_sel = jnp.dot(onehot0, lv_ref[...], preferred_element_type=jnp.float32,
                         precision=HI)
        q_sel = jnp.dot(onehot0, qkv_s[:, 0:D], preferred_element_type=jnp.float32,
                        precision=HI)

        k_all = qkv_s[:, D:2 * D]
        v_all = qkv_s[:, 2 * D:3 * D]
        ctxs = []
        for h in range(H):
            s = jax.lax.dot_general(q_sel[:, h * HD:(h + 1) * HD], k_all[:, h * HD:(h + 1) * HD],
                                    (((1,), (1,)), ((), ())),
                                    preferred_element_type=jnp.float32) * scale
            s = s - _rowmax(s)
            e = jnp.exp(s)
            p = e / jnp.sum(e, axis=1, keepdims=True)
            ctxs.append(jnp.dot(p, v_all[:, h * HD:(h + 1) * HD],
                                preferred_element_type=jnp.float32))
        ctx = jnp.concatenate(ctxs, axis=1)
        mha = jnp.dot(ctx, wo_ref[...], preferred_element_type=jnp.float32) + bo_ref[...]
        x0 = _ln(lv_sel + mha, ln1g_ref[...], ln1b_ref[...])

        red18 = jnp.where(jax.lax.broadcasted_iota(jnp.int32, (18, 90), 0) ==
                          (jax.lax.broadcasted_iota(jnp.int32, (18, 90), 1) // KN),
                          1.0, 0.0)
        agg1 = jnp.dot(red18, x0[18:108, :] * norm1, preferred_element_type=jnp.float32,
                       precision=HI)
        x1 = _ln(x0[0:18, :] + jnp.dot(agg1, wg0_ref[...], preferred_element_type=jnp.float32) + bg0_ref[...],
                 lng0_ref[...], lnb0_ref[...])

        red3 = jnp.where(jax.lax.broadcasted_iota(jnp.int32, (3, 15), 0) ==
                         (jax.lax.broadcasted_iota(jnp.int32, (3, 15), 1) // KN),
                         1.0, 0.0)
        agg2 = jnp.dot(red3, x1[3:18, :] * norm2, preferred_element_type=jnp.float32,
                       precision=HI)
        x2 = _ln(x1[0:3, :] + jnp.dot(agg2, wg1_ref[...], preferred_element_type=jnp.float32) + bg1_ref[...],
                 lng1_ref[...], lnb1_ref[...])

        for row, (w1, b1, w2, b2, out) in enumerate((
                (w1m_ref, b1m_ref, w2m_ref, b2m_ref, o1_ref),
                (w1y_ref, b1y_ref, w2y_ref, b2y_ref, o2_ref),
                (w1r_ref, b1r_ref, w2r_ref, b2r_ref, o3_ref))):
            hdn = jnp.maximum(
                jnp.dot(x2[row:row + 1, :], w1[...], preferred_element_type=jnp.float32) + b1[...],
                0.0)
            out[...] = jnp.dot(hdn, w2[...], preferred_element_type=jnp.float32) + b2[...]


def _full(shp):
    return pl.BlockSpec(shp, lambda i: tuple(0 for _ in shp))


@jax.jit
def kernel(latent_vectors, Wq, bq, Wk, bk, Wv, bv, Wo, bo, ln1_g, ln1_b,
           Wg0, bg0, lng0, lnb0, Wg1, bg1, lng1, lnb1,
           W1_mrna, b1_mrna, W2_mrna, b2_mrna,
           W1_methylation, b1_methylation, W2_methylation, b2_methylation,
           W1_mirna, b1_mirna, W2_mirna, b2_mirna):
    in_specs = [_full((N, D)),
                _full((D, D)), _full((D, D)), _full((D, D)),
                _full((1, D)), _full((1, D)), _full((1, D)),
                _full((D, D)), _full((1, D)), _full((1, D)), _full((1, D)),
                _full((D, D)), _full((1, D)), _full((1, D)), _full((1, D)),
                _full((D, D)), _full((1, D)), _full((1, D)), _full((1, D)),
                _full((D, 256)), _full((1, 256)), _full((256, 1000)), _full((1, 1000)),
                _full((D, 256)), _full((1, 256)), _full((256, 2000)), _full((1, 2000)),
                _full((D, 256)), _full((1, 256)), _full((256, 500)), _full((1, 500))]
    o1, o2, o3 = pl.pallas_call(
        _body,
        grid=(NB + 2,),
        in_specs=in_specs,
        out_specs=[_full((1, 1000)), _full((1, 2000)), _full((1, 500))],
        out_shape=[jax.ShapeDtypeStruct((1, 1000), jnp.float32),
                   jax.ShapeDtypeStruct((1, 2000), jnp.float32),
                   jax.ShapeDtypeStruct((1, 500), jnp.float32)],
        scratch_shapes=[pltpu.VMEM((N, 3 * D), jnp.float32),
                        pltpu.VMEM((N, KN), jnp.float32)],
    )(latent_vectors,
      Wq.reshape(D, D), Wk.reshape(D, D), Wv.reshape(D, D),
      bq.reshape(1, D), bk.reshape(1, D), bv.reshape(1, D),
      Wo.reshape(D, D), bo.reshape(1, D), ln1_g.reshape(1, D), ln1_b.reshape(1, D),
      Wg0, bg0.reshape(1, D), lng0.reshape(1, D), lnb0.reshape(1, D),
      Wg1, bg1.reshape(1, D), lng1.reshape(1, D), lnb1.reshape(1, D),
      W1_mrna, b1_mrna.reshape(1, -1), W2_mrna, b2_mrna.reshape(1, -1),
      W1_methylation, b1_methylation.reshape(1, -1), W2_methylation, b2_methylation.reshape(1, -1),
      W1_mirna, b1_mirna.reshape(1, -1), W2_mirna, b2_mirna.reshape(1, -1))
    return (o1.reshape(-1), o2.reshape(-1), o3.reshape(-1))
